# baseline (speedup 1.0000x reference)
import functools

import jax
import jax.numpy as jnp
from jax import lax
from jax.experimental import pallas as pl
from jax.experimental.pallas import tpu as pltpu
from jax.experimental.pallas import tpu_sc as plsc

_N = 640000
_NSEG = 10000
_H = 128
_R = 5120
_NBLK = _N // _R
_Q = 80
_SPAD = _Q * 128
_TAU = 0.1
_EPS = 1e-5


def _ln_cols(t, g, b):
    mu = jnp.mean(t, axis=0, keepdims=True)
    d = t - mu
    var = jnp.mean(d * d, axis=0, keepdims=True)
    return d / jnp.sqrt(var + _EPS) * g + b


def _tc_body(x_ref, b_ref, w1_ref, b1_ref, g1_ref, be1_ref,
             w2_ref, b2_ref, g2_ref, be2_ref, w3_ref, b3_ref, w4_ref,
             b4_ref, w5_ref, g0_ref, g1n_ref, dec_ref,
             acc_ref, comp_ref, cnt_ref, scal_ref):
    pid = pl.program_id(0)
    x = x_ref[0]
    bi = b_ref[0]
    t = w1_ref[...] * x + b1_ref[...]
    h1 = jax.nn.relu(_ln_cols(t, g1_ref[...], be1_ref[...]))
    h2 = jnp.dot(w2_ref[...], h1, preferred_element_type=jnp.float32)
    h2 = jax.nn.relu(_ln_cols(h2 + b2_ref[...], g2_ref[...], be2_ref[...]))
    u = jax.nn.relu(
        jnp.dot(w3_ref[...], h2, preferred_element_type=jnp.float32)
        + b3_ref[...])
    a = jnp.dot(w4_ref[...], u, preferred_element_type=jnp.float32) \
        + b4_ref[...]
    z = jnp.dot(w5_ref[...], h2, preferred_element_type=jnp.float32)

    first = pid == 0
    blkmax = jnp.max(a)
    m_old = jnp.where(first, blkmax, scal_ref[0, 0])
    m_new = jnp.maximum(m_old, blkmax)
    alpha = jnp.exp(m_old - m_new)
    e = jnp.exp(a - m_new)

    w0 = z[0:1] * e
    w1v = z[1:2] * e
    q_oh = (bi // 128 ==
            lax.broadcasted_iota(jnp.int32, (_Q, _R), 0)
            ).astype(jnp.float32)
    r_oh = (bi % 128 ==
            lax.broadcasted_iota(jnp.int32, (128, _R), 0)
            ).astype(jnp.float32)
    lhs = jnp.concatenate([q_oh * w0, q_oh * w1v], axis=0)
    part = jax.lax.dot_general(
        lhs, r_oh, (((1,), (1,)), ((), ())),
        preferred_element_type=jnp.float32)
    part_c = jax.lax.dot_general(
        q_oh, r_oh, (((1,), (1,)), ((), ())),
        preferred_element_type=jnp.float32)

    acc_old = jnp.where(first, 0.0, acc_ref[...]) * alpha
    comp_old = jnp.where(first, 0.0, comp_ref[...]) * alpha
    y = part - comp_old
    t_acc = acc_old + y
    comp_ref[...] = (t_acc - acc_old) - y
    acc_ref[...] = t_acc
    cnt_ref[...] = jnp.where(first, 0.0, cnt_ref[...]) + part_c

    s_old = jnp.where(first, 0.0, scal_ref[1, 0]) * alpha
    cs_old = jnp.where(first, 0.0, scal_ref[2, 0]) * alpha
    ys = jnp.sum(e) - cs_old
    t_s = s_old + ys
    scal_ref[0, 0] = m_new
    scal_ref[1, 0] = t_s
    scal_ref[2, 0] = (t_s - s_old) - ys

    @pl.when(pid == _NBLK - 1)
    def _():
        acc = acc_ref[...]
        s_tot = scal_ref[1, 0]
        po0 = acc[0:_Q, :]
        po1 = acc[_Q:2 * _Q, :]
        cnt = jnp.clip(cnt_ref[...], 1.0, None)
        l0 = po0 / (s_tot * cnt)
        l1 = po1 / (s_tot * cnt)
        y0 = (l0 + g0_ref[...]) / _TAU
        y1 = (l1 + g1n_ref[...]) / _TAU
        mm = jnp.maximum(y0, y1)
        e0 = jnp.exp(y0 - mm)
        e1 = jnp.exp(y1 - mm)
        ys0 = e0 / (e0 + e1)
        hard0 = jnp.where(y0 >= y1, 1.0, 0.0)
        dec_ref[...] = (hard0 - ys0) + ys0


def _tc_pooled(xrow, brow, w1t, b1t, g1t, be1t, w2t, b2t, g2t, be2t,
               w3t, b3t, w4t, b4t, w5t, gum0, gum1):
    full = lambda s: pl.BlockSpec(s, lambda i: (0,) * len(s))
    return pl.pallas_call(
        _tc_body,
        grid=(_NBLK,),
        in_specs=[
            pl.BlockSpec((1, 1, _R), lambda i: (i, 0, 0)),
            pl.BlockSpec((1, 1, _R), lambda i: (i, 0, 0)),
            full((_H, _R)), full((_H, _R)), full((_H, _R)), full((_H, _R)),
            full((_H, _H)), full((_H, _R)), full((_H, _R)), full((_H, _R)),
            full((_H // 2, _H)), full((_H // 2, _R)),
            full((1, _H // 2)), full((1, 1)),
            full((2, _H)),
            full((_Q, 128)), full((_Q, 128)),
        ],
        out_specs=pl.BlockSpec((_Q, 128), lambda i: (0, 0)),
        out_shape=jax.ShapeDtypeStruct((_Q, 128), jnp.float32),
        scratch_shapes=[
            pltpu.VMEM((2 * _Q, 128), jnp.float32),
            pltpu.VMEM((2 * _Q, 128), jnp.float32),
            pltpu.VMEM((_Q, 128), jnp.float32),
            pltpu.SMEM((3, 1), jnp.float32),
        ],
    )(xrow, brow, w1t, b1t, g1t, be1t, w2t, b2t, g2t, be2t, w3t, b3t,
      w4t, b4t, w5t, gum0, gum1)


_NW = 32
_CH = _N // _NW


def _sc_gather(table, idx):
    mesh = plsc.VectorSubcoreMesh(core_axis_name="c", subcore_axis_name="s")

    @functools.partial(
        pl.kernel, mesh=mesh,
        out_type=jax.ShapeDtypeStruct((_N,), jnp.float32),
        compiler_params=pltpu.CompilerParams(needs_layout_passes=False),
        scratch_types=[
            pltpu.VMEM((_SPAD,), jnp.float32),
            pltpu.VMEM((_CH,), jnp.int32),
            pltpu.VMEM((_CH,), jnp.float32),
        ],
    )
    def k(tbl_hbm, idx_hbm, out_hbm, tbl_v, idx_v, val_v):
        wid = lax.axis_index("s") * 2 + lax.axis_index("c")
        base = wid * _CH
        pltpu.sync_copy(tbl_hbm, tbl_v)
        pltpu.sync_copy(idx_hbm.at[pl.ds(base, _CH)], idx_v)

        def body(i, carry):
            ix = idx_v[pl.ds(i * 16, 16)]
            val_v[pl.ds(i * 16, 16)] = plsc.load_gather(tbl_v, [ix])
            return carry

        lax.fori_loop(0, _CH // 16, body, 0, unroll=8)
        pltpu.sync_copy(val_v, out_hbm.at[pl.ds(base, _CH)])

    return k(table, idx)


def kernel(x, batch, W1, b1, g1, be1, W2, b2, g2, be2, W3, b3, W4, b4,
           W5, b5):
    xrow = x.astype(jnp.float32).reshape(_NBLK, 1, _R)
    bi = batch.astype(jnp.int32)
    brow = bi.reshape(_NBLK, 1, _R)

    tile = lambda v: jnp.broadcast_to(
        v.astype(jnp.float32).reshape(-1, 1), (v.size, _R))

    gk = jax.random.key(42)
    u = jax.random.uniform(gk, (_NSEG, 2), jnp.float32, 1e-10, 1.0)
    gn = -jnp.log(-jnp.log(u)) + b5[None, :]
    gn = jnp.pad(gn, ((0, _SPAD - _NSEG), (0, 0)))
    gum0 = gn[:, 0].reshape(_Q, 128)
    gum1 = gn[:, 1].reshape(_Q, 128)

    dec = _tc_pooled(
        xrow, brow,
        tile(W1), tile(b1), tile(g1), tile(be1),
        W2.T, tile(b2), tile(g2), tile(be2),
        W3.T, tile(b3), W4.T, b4.reshape(1, 1), W5.T, gum0, gum1)

    table = dec.reshape(_SPAD)
    return _sc_gather(table, bi)


# speedup vs baseline: 1.1433x; 1.1433x over previous
import functools

import jax
import jax.numpy as jnp
from jax import lax
from jax.experimental import pallas as pl
from jax.experimental.pallas import tpu as pltpu
from jax.experimental.pallas import tpu_sc as plsc

_N = 640000
_NSEG = 10000
_H = 128
_R = 5120
_NBLK = _N // _R
_Q = 80
_SPAD = _Q * 128
_TAU = 0.1
_EPS = 1e-5


def _ln_cols(t, g, b):
    mu = jnp.mean(t, axis=0, keepdims=True)
    d = t - mu
    var = jnp.mean(d * d, axis=0, keepdims=True)
    return d / jnp.sqrt(var + _EPS) * g + b


def _tc_body(x_ref, b_ref, gw_ref, gb_ref, be1_ref, pp_ref, pq_ref,
             pc_ref,
             w2_ref, b2_ref, g2_ref, g2b_ref, w3_ref, b3_ref, w4_ref,
             b4_ref, w5_ref, g0_ref, g1n_ref, dec_ref,
             acc_ref, comp_ref, cnt_ref, scal_ref):
    pid = pl.program_id(0)
    x = x_ref[0]
    bi = b_ref[0]
    var1 = pp_ref[...] * (x * x) + pq_ref[...] * x + pc_ref[...]
    inv1 = 1.0 / jnp.sqrt(var1 + _EPS)
    h1 = jax.nn.relu(
        gw_ref[...] * (x * inv1) + gb_ref[...] * inv1 + be1_ref[...])
    h2 = jnp.dot(w2_ref[...], h1, preferred_element_type=jnp.float32)
    h2 = jax.nn.relu(_ln_cols(h2 + b2_ref[...], g2_ref[...], g2b_ref[...]))
    u = jax.nn.relu(
        jnp.dot(w3_ref[...], h2, preferred_element_type=jnp.float32)
        + b3_ref[...])
    a = jnp.dot(w4_ref[...], u, preferred_element_type=jnp.float32) \
        + b4_ref[...]
    z = jnp.dot(w5_ref[...], h2, preferred_element_type=jnp.float32)

    first = pid == 0
    blkmax = jnp.max(a)
    m_old = jnp.where(first, blkmax, scal_ref[0, 0])
    m_new = jnp.maximum(m_old, blkmax)
    alpha = jnp.exp(m_old - m_new)
    e = jnp.exp(a - m_new)

    w0 = z[0:1] * e
    w1v = z[1:2] * e
    q_oh = (bi // 128 ==
            lax.broadcasted_iota(jnp.int32, (_Q, _R), 0)
            ).astype(jnp.float32)
    r_oh = (bi % 128 ==
            lax.broadcasted_iota(jnp.int32, (128, _R), 0)
            ).astype(jnp.float32)
    lhs = jnp.concatenate([q_oh * w0, q_oh * w1v], axis=0)
    part = jax.lax.dot_general(
        lhs, r_oh, (((1,), (1,)), ((), ())),
        preferred_element_type=jnp.float32)
    part_c = jax.lax.dot_general(
        q_oh, r_oh, (((1,), (1,)), ((), ())),
        preferred_element_type=jnp.float32)

    acc_old = jnp.where(first, 0.0, acc_ref[...]) * alpha
    comp_old = jnp.where(first, 0.0, comp_ref[...]) * alpha
    y = part - comp_old
    t_acc = acc_old + y
    comp_ref[...] = (t_acc - acc_old) - y
    acc_ref[...] = t_acc
    cnt_ref[...] = jnp.where(first, 0.0, cnt_ref[...]) + part_c

    s_old = jnp.where(first, 0.0, scal_ref[1, 0]) * alpha
    cs_old = jnp.where(first, 0.0, scal_ref[2, 0]) * alpha
    ys = jnp.sum(e) - cs_old
    t_s = s_old + ys
    scal_ref[0, 0] = m_new
    scal_ref[1, 0] = t_s
    scal_ref[2, 0] = (t_s - s_old) - ys

    @pl.when(pid == _NBLK - 1)
    def _():
        acc = acc_ref[...]
        s_tot = scal_ref[1, 0]
        po0 = acc[0:_Q, :]
        po1 = acc[_Q:2 * _Q, :]
        cnt = jnp.clip(cnt_ref[...], 1.0, None)
        l0 = po0 / (s_tot * cnt)
        l1 = po1 / (s_tot * cnt)
        y0 = (l0 + g0_ref[...]) / _TAU
        y1 = (l1 + g1n_ref[...]) / _TAU
        mm = jnp.maximum(y0, y1)
        e0 = jnp.exp(y0 - mm)
        e1 = jnp.exp(y1 - mm)
        ys0 = e0 / (e0 + e1)
        hard0 = jnp.where(y0 >= y1, 1.0, 0.0)
        dec_ref[...] = (hard0 - ys0) + ys0


def _tc_pooled(xrow, brow, gwt, gbt, be1t, pp, pq, pc, w2t, b2t, g2t,
               be2t, w3t, b3t, w4t, b4t, w5t, gum0, gum1):
    full = lambda s: pl.BlockSpec(s, lambda i: (0,) * len(s))
    return pl.pallas_call(
        _tc_body,
        grid=(_NBLK,),
        in_specs=[
            pl.BlockSpec((1, 1, _R), lambda i: (i, 0, 0)),
            pl.BlockSpec((1, 1, _R), lambda i: (i, 0, 0)),
            full((_H, _R)), full((_H, _R)), full((_H, _R)),
            full((1, 1)), full((1, 1)), full((1, 1)),
            full((_H, _H)), full((_H, _R)), full((_H, _R)), full((_H, _R)),
            full((_H // 2, _H)), full((_H // 2, _R)),
            full((1, _H // 2)), full((1, 1)),
            full((2, _H)),
            full((_Q, 128)), full((_Q, 128)),
        ],
        out_specs=pl.BlockSpec((_Q, 128), lambda i: (0, 0)),
        out_shape=jax.ShapeDtypeStruct((_Q, 128), jnp.float32),
        scratch_shapes=[
            pltpu.VMEM((2 * _Q, 128), jnp.float32),
            pltpu.VMEM((2 * _Q, 128), jnp.float32),
            pltpu.VMEM((_Q, 128), jnp.float32),
            pltpu.SMEM((3, 1), jnp.float32),
        ],
    )(xrow, brow, gwt, gbt, be1t, pp, pq, pc, w2t, b2t, g2t, be2t,
      w3t, b3t, w4t, b4t, w5t, gum0, gum1)


_NW = 32
_CH = _N // _NW


def _sc_gather(table, idx):
    mesh = plsc.VectorSubcoreMesh(core_axis_name="c", subcore_axis_name="s")

    @functools.partial(
        pl.kernel, mesh=mesh,
        out_type=jax.ShapeDtypeStruct((_N,), jnp.float32),
        compiler_params=pltpu.CompilerParams(needs_layout_passes=False),
        scratch_types=[
            pltpu.VMEM((_SPAD,), jnp.float32),
            pltpu.VMEM((_CH,), jnp.int32),
            pltpu.VMEM((_CH,), jnp.float32),
        ],
    )
    def k(tbl_hbm, idx_hbm, out_hbm, tbl_v, idx_v, val_v):
        wid = lax.axis_index("s") * 2 + lax.axis_index("c")
        base = wid * _CH
        pltpu.sync_copy(tbl_hbm, tbl_v)
        pltpu.sync_copy(idx_hbm.at[pl.ds(base, _CH)], idx_v)

        def body(i, carry):
            ix = idx_v[pl.ds(i * 16, 16)]
            val_v[pl.ds(i * 16, 16)] = plsc.load_gather(tbl_v, [ix])
            return carry

        lax.fori_loop(0, _CH // 16, body, 0, unroll=8)
        pltpu.sync_copy(val_v, out_hbm.at[pl.ds(base, _CH)])

    return k(table, idx)


def kernel(x, batch, W1, b1, g1, be1, W2, b2, g2, be2, W3, b3, W4, b4,
           W5, b5):
    xrow = x.astype(jnp.float32).reshape(_NBLK, 1, _R)
    bi = batch.astype(jnp.int32)
    brow = bi.reshape(_NBLK, 1, _R)

    tile = lambda v: jnp.broadcast_to(
        v.astype(jnp.float32).reshape(-1, 1), (v.size, _R))

    gk = jax.random.key(42)
    u = jax.random.uniform(gk, (_NSEG, 2), jnp.float32, 1e-10, 1.0)
    gn = -jnp.log(-jnp.log(u)) + b5[None, :]
    gn = jnp.pad(gn, ((0, _SPAD - _NSEG), (0, 0)))
    gum0 = gn[:, 0].reshape(_Q, 128)
    gum1 = gn[:, 1].reshape(_Q, 128)

    w1f = W1.reshape(_H).astype(jnp.float32)
    b1f = b1.astype(jnp.float32)
    wc = w1f - jnp.mean(w1f)
    bc = b1f - jnp.mean(b1f)
    pp = jnp.mean(wc * wc).reshape(1, 1)
    pq = (2.0 * jnp.mean(wc * bc)).reshape(1, 1)
    pc = jnp.mean(bc * bc).reshape(1, 1)

    dec = _tc_pooled(
        xrow, brow,
        tile(g1 * wc), tile(g1 * bc), tile(be1), pp, pq, pc,
        W2.T, tile(b2), tile(g2), tile(be2),
        W3.T, tile(b3), W4.T, b4.reshape(1, 1), W5.T, gum0, gum1)

    table = dec.reshape(_SPAD)
    return _sc_gather(table, bi)


# speedup vs baseline: 1.1551x; 1.0103x over previous
import functools

import jax
import jax.numpy as jnp
from jax import lax
from jax.experimental import pallas as pl
from jax.experimental.pallas import tpu as pltpu
from jax.experimental.pallas import tpu_sc as plsc

_N = 640000
_NSEG = 10000
_H = 128
_R = 6400
_NBLK = _N // _R
_Q = 80
_SPAD = _Q * 128
_TAU = 0.1
_EPS = 1e-5


def _ln_cols(t, g, b):
    mu = jnp.mean(t, axis=0, keepdims=True)
    d = t - mu
    var = jnp.mean(d * d, axis=0, keepdims=True)
    return d / jnp.sqrt(var + _EPS) * g + b


def _tc_body(x_ref, b_ref, gw_ref, gb_ref, be1_ref, pp_ref, pq_ref,
             pc_ref,
             w2_ref, b2_ref, g2_ref, g2b_ref, w3_ref, b3_ref, w4_ref,
             b4_ref, w5_ref, g0_ref, g1n_ref, dec_ref,
             acc_ref, comp_ref, cnt_ref, scal_ref):
    pid = pl.program_id(0)
    x = x_ref[0]
    bi = b_ref[0]
    var1 = pp_ref[...] * (x * x) + pq_ref[...] * x + pc_ref[...]
    inv1 = 1.0 / jnp.sqrt(var1 + _EPS)
    h1 = jax.nn.relu(
        gw_ref[...] * (x * inv1) + gb_ref[...] * inv1 + be1_ref[...])
    h2 = jnp.dot(w2_ref[...], h1, preferred_element_type=jnp.float32)
    h2 = jax.nn.relu(_ln_cols(h2 + b2_ref[...], g2_ref[...], g2b_ref[...]))
    u = jax.nn.relu(
        jnp.dot(w3_ref[...], h2, preferred_element_type=jnp.float32)
        + b3_ref[...])
    a = jnp.dot(w4_ref[...], u, preferred_element_type=jnp.float32) \
        + b4_ref[...]
    z = jnp.dot(w5_ref[...], h2, preferred_element_type=jnp.float32)

    first = pid == 0
    blkmax = jnp.max(a)
    m_old = jnp.where(first, blkmax, scal_ref[0, 0])
    m_new = jnp.maximum(m_old, blkmax)
    alpha = jnp.exp(m_old - m_new)
    e = jnp.exp(a - m_new)

    w0 = z[0:1] * e
    w1v = z[1:2] * e
    q_oh = (bi // 128 ==
            lax.broadcasted_iota(jnp.int32, (_Q, _R), 0)
            ).astype(jnp.float32)
    r_oh = (bi % 128 ==
            lax.broadcasted_iota(jnp.int32, (128, _R), 0)
            ).astype(jnp.float32)
    lhs = jnp.concatenate([q_oh * w0, q_oh * w1v], axis=0)
    part = jax.lax.dot_general(
        lhs, r_oh, (((1,), (1,)), ((), ())),
        preferred_element_type=jnp.float32)
    part_c = jax.lax.dot_general(
        q_oh, r_oh, (((1,), (1,)), ((), ())),
        preferred_element_type=jnp.float32)

    acc_old = jnp.where(first, 0.0, acc_ref[...]) * alpha
    comp_old = jnp.where(first, 0.0, comp_ref[...]) * alpha
    y = part - comp_old
    t_acc = acc_old + y
    comp_ref[...] = (t_acc - acc_old) - y
    acc_ref[...] = t_acc
    cnt_ref[...] = jnp.where(first, 0.0, cnt_ref[...]) + part_c

    s_old = jnp.where(first, 0.0, scal_ref[1, 0]) * alpha
    cs_old = jnp.where(first, 0.0, scal_ref[2, 0]) * alpha
    ys = jnp.sum(e) - cs_old
    t_s = s_old + ys
    scal_ref[0, 0] = m_new
    scal_ref[1, 0] = t_s
    scal_ref[2, 0] = (t_s - s_old) - ys

    @pl.when(pid == _NBLK - 1)
    def _():
        acc = acc_ref[...]
        s_tot = scal_ref[1, 0]
        po0 = acc[0:_Q, :]
        po1 = acc[_Q:2 * _Q, :]
        cnt = jnp.clip(cnt_ref[...], 1.0, None)
        l0 = po0 / (s_tot * cnt)
        l1 = po1 / (s_tot * cnt)
        y0 = (l0 + g0_ref[...]) / _TAU
        y1 = (l1 + g1n_ref[...]) / _TAU
        mm = jnp.maximum(y0, y1)
        e0 = jnp.exp(y0 - mm)
        e1 = jnp.exp(y1 - mm)
        ys0 = e0 / (e0 + e1)
        hard0 = jnp.where(y0 >= y1, 1.0, 0.0)
        dec_ref[...] = (hard0 - ys0) + ys0


def _tc_pooled(xrow, brow, gwt, gbt, be1t, pp, pq, pc, w2t, b2t, g2t,
               be2t, w3t, b3t, w4t, b4t, w5t, gum0, gum1):
    full = lambda s: pl.BlockSpec(s, lambda i: (0,) * len(s))
    return pl.pallas_call(
        _tc_body,
        grid=(_NBLK,),
        in_specs=[
            pl.BlockSpec((1, 1, _R), lambda i: (i, 0, 0)),
            pl.BlockSpec((1, 1, _R), lambda i: (i, 0, 0)),
            full((_H, _R)), full((_H, _R)), full((_H, _R)),
            full((1, 1)), full((1, 1)), full((1, 1)),
            full((_H, _H)), full((_H, _R)), full((_H, _R)), full((_H, _R)),
            full((_H // 2, _H)), full((_H // 2, _R)),
            full((1, _H // 2)), full((1, 1)),
            full((2, _H)),
            full((_Q, 128)), full((_Q, 128)),
        ],
        out_specs=pl.BlockSpec((_Q, 128), lambda i: (0, 0)),
        out_shape=jax.ShapeDtypeStruct((_Q, 128), jnp.float32),
        scratch_shapes=[
            pltpu.VMEM((2 * _Q, 128), jnp.float32),
            pltpu.VMEM((2 * _Q, 128), jnp.float32),
            pltpu.VMEM((_Q, 128), jnp.float32),
            pltpu.SMEM((3, 1), jnp.float32),
        ],
    )(xrow, brow, gwt, gbt, be1t, pp, pq, pc, w2t, b2t, g2t, be2t,
      w3t, b3t, w4t, b4t, w5t, gum0, gum1)


_NW = 32
_CH = _N // _NW


def _sc_gather(table, idx):
    mesh = plsc.VectorSubcoreMesh(core_axis_name="c", subcore_axis_name="s")

    @functools.partial(
        pl.kernel, mesh=mesh,
        out_type=jax.ShapeDtypeStruct((_N,), jnp.float32),
        compiler_params=pltpu.CompilerParams(needs_layout_passes=False),
        scratch_types=[
            pltpu.VMEM((_SPAD,), jnp.float32),
            pltpu.VMEM((_CH,), jnp.int32),
            pltpu.VMEM((_CH,), jnp.float32),
        ],
    )
    def k(tbl_hbm, idx_hbm, out_hbm, tbl_v, idx_v, val_v):
        wid = lax.axis_index("s") * 2 + lax.axis_index("c")
        base = wid * _CH
        pltpu.sync_copy(tbl_hbm, tbl_v)
        pltpu.sync_copy(idx_hbm.at[pl.ds(base, _CH)], idx_v)

        def body(i, carry):
            ix = idx_v[pl.ds(i * 16, 16)]
            val_v[pl.ds(i * 16, 16)] = plsc.load_gather(tbl_v, [ix])
            return carry

        lax.fori_loop(0, _CH // 16, body, 0, unroll=8)
        pltpu.sync_copy(val_v, out_hbm.at[pl.ds(base, _CH)])

    return k(table, idx)


def kernel(x, batch, W1, b1, g1, be1, W2, b2, g2, be2, W3, b3, W4, b4,
           W5, b5):
    xrow = x.astype(jnp.float32).reshape(_NBLK, 1, _R)
    bi = batch.astype(jnp.int32)
    brow = bi.reshape(_NBLK, 1, _R)

    tile = lambda v: jnp.broadcast_to(
        v.astype(jnp.float32).reshape(-1, 1), (v.size, _R))

    gk = jax.random.key(42)
    u = jax.random.uniform(gk, (_NSEG, 2), jnp.float32, 1e-10, 1.0)
    gn = -jnp.log(-jnp.log(u)) + b5[None, :]
    gn = jnp.pad(gn, ((0, _SPAD - _NSEG), (0, 0)))
    gum0 = gn[:, 0].reshape(_Q, 128)
    gum1 = gn[:, 1].reshape(_Q, 128)

    w1f = W1.reshape(_H).astype(jnp.float32)
    b1f = b1.astype(jnp.float32)
    wc = w1f - jnp.mean(w1f)
    bc = b1f - jnp.mean(b1f)
    pp = jnp.mean(wc * wc).reshape(1, 1)
    pq = (2.0 * jnp.mean(wc * bc)).reshape(1, 1)
    pc = jnp.mean(bc * bc).reshape(1, 1)

    dec = _tc_pooled(
        xrow, brow,
        tile(g1 * wc), tile(g1 * bc), tile(be1), pp, pq, pc,
        W2.T, tile(b2), tile(g2), tile(be2),
        W3.T, tile(b3), W4.T, b4.reshape(1, 1), W5.T, gum0, gum1)

    table = dec.reshape(_SPAD)
    return _sc_gather(table, bi)
